# baseline (speedup 1.0000x reference)
import functools

import jax
import jax.numpy as jnp
from jax.experimental import pallas as pl

_N_NODES = 39
_HIDDEN = 128
_CONT = 13
_CATF = 26


def _main_body(num_ref, cat_ref, embw_ref, convw_ref, h_ref):
    i = pl.program_id(0)
    emb = jnp.sum(cat_ref[...] * embw_ref[...][:, None, :], axis=2)
    x = jnp.concatenate([num_ref[...], emb], axis=0)
    acc = jax.lax.dot_general(
        x, convw_ref[...], (((1,), (0,)), ((), ())),
        preferred_element_type=jnp.float32)

    @pl.when(i == 0)
    def _():
        h_ref[...] = acc

    @pl.when(i > 0)
    def _():
        h_ref[...] += acc


def _epilogue_body(h_ref, ei_ref, ew_ref, van_ref, fcw_ref, fcb_ref, out_ref):
    src = ei_ref[0, :]
    dst = ei_ref[1, :]
    w = ew_ref[0, :]
    e = src.shape[0]
    n = _N_NODES
    node_ids = jax.lax.broadcasted_iota(jnp.int32, (e, n), 1)
    osrc = (src[:, None] == node_ids).astype(jnp.float32)
    odst = (dst[:, None] == node_ids).astype(jnp.float32)
    deg = 1.0 + jnp.sum(odst * w[:, None], axis=0)
    dinv = jnp.where(deg > 0, jax.lax.rsqrt(deg), 0.0)
    dinv_src = jnp.sum(osrc * dinv[None, :], axis=1)
    dinv_dst = jnp.sum(odst * dinv[None, :], axis=1)
    norm = dinv_src * w * dinv_dst
    a = jax.lax.dot_general(
        odst * norm[:, None], osrc, (((0,), (0,)), ((), ())),
        preferred_element_type=jnp.float32)
    rows = jax.lax.broadcasted_iota(jnp.int32, (n, n), 0)
    cols = jax.lax.broadcasted_iota(jnp.int32, (n, n), 1)
    a = a + jnp.where(rows == cols, dinv[:, None] * dinv[None, :], 0.0)
    hn = jax.nn.relu(
        jax.lax.dot_general(a, h_ref[...], (((1,), (0,)), ((), ())),
                            preferred_element_type=jnp.float32))
    pooled = jnp.sum(hn, axis=0) / jnp.float32(n)
    c = jnp.sum(pooled * fcw_ref[_NUM_CLASSES:, 0]) + fcb_ref[0, 0]
    z = jax.lax.dot_general(
        van_ref[...], fcw_ref[: _NUM_CLASSES, :], (((1,), (0,)), ((), ())),
        preferred_element_type=jnp.float32) + c
    beta = jnp.float32(1.1)
    bz = beta * z
    t = (jnp.maximum(bz, 0.0) + jnp.log1p(jnp.exp(-jnp.abs(bz)))) / beta
    out_ref[...] = van_ref[...] / t


_NUM_CLASSES = 10


@jax.jit
def kernel(num_x, cat_x, edge_index, edge_weights, batch, vanilla_out,
           emb_W, conv_W, fc_W, fc_b):
    del batch
    nf = num_x.shape[1]
    tile = 1024
    grid = nf // tile
    h = pl.pallas_call(
        _main_body,
        grid=(grid,),
        in_specs=[
            pl.BlockSpec((_CONT, tile), lambda i: (0, i)),
            pl.BlockSpec((_CATF, tile, _HIDDEN), lambda i: (0, i, 0)),
            pl.BlockSpec((_CATF, _HIDDEN), lambda i: (0, 0)),
            pl.BlockSpec((tile, _HIDDEN), lambda i: (i, 0)),
        ],
        out_specs=pl.BlockSpec((_N_NODES, _HIDDEN), lambda i: (0, 0)),
        out_shape=jax.ShapeDtypeStruct((_N_NODES, _HIDDEN), jnp.float32),
    )(num_x, cat_x, emb_W, conv_W)

    out = pl.pallas_call(
        _epilogue_body,
        out_shape=jax.ShapeDtypeStruct(vanilla_out.shape, jnp.float32),
    )(h, edge_index, edge_weights.reshape(1, -1), vanilla_out, fc_W,
      fc_b.reshape(1, 1))
    return out


# speedup vs baseline: 1.0143x; 1.0143x over previous
import functools

import jax
import jax.numpy as jnp
from jax.experimental import pallas as pl

_N_NODES = 39
_HIDDEN = 128
_CONT = 13
_CATF = 26


def _main_body(num_ref, cat_ref, embw_ref, convw_ref, h_ref):
    i = pl.program_id(0)
    emb = jnp.sum(cat_ref[...] * embw_ref[...][:, None, :], axis=2)
    x = jnp.concatenate([num_ref[...], emb], axis=0)
    acc = jax.lax.dot_general(
        x, convw_ref[...], (((1,), (0,)), ((), ())),
        preferred_element_type=jnp.float32)

    @pl.when(i == 0)
    def _():
        h_ref[...] = acc

    @pl.when(i > 0)
    def _():
        h_ref[...] += acc


def _epilogue_body(h_ref, ei_ref, ew_ref, van_ref, fcw_ref, fcb_ref, out_ref):
    src = ei_ref[0, :]
    dst = ei_ref[1, :]
    w = ew_ref[0, :]
    e = src.shape[0]
    n = _N_NODES
    node_ids = jax.lax.broadcasted_iota(jnp.int32, (e, n), 1)
    osrc = (src[:, None] == node_ids).astype(jnp.float32)
    odst = (dst[:, None] == node_ids).astype(jnp.float32)
    deg = 1.0 + jnp.sum(odst * w[:, None], axis=0)
    dinv = jnp.where(deg > 0, jax.lax.rsqrt(deg), 0.0)
    dinv_src = jnp.sum(osrc * dinv[None, :], axis=1)
    dinv_dst = jnp.sum(odst * dinv[None, :], axis=1)
    norm = dinv_src * w * dinv_dst
    a = jax.lax.dot_general(
        odst * norm[:, None], osrc, (((0,), (0,)), ((), ())),
        preferred_element_type=jnp.float32)
    rows = jax.lax.broadcasted_iota(jnp.int32, (n, n), 0)
    cols = jax.lax.broadcasted_iota(jnp.int32, (n, n), 1)
    a = a + jnp.where(rows == cols, dinv[:, None] * dinv[None, :], 0.0)
    hn = jax.nn.relu(
        jax.lax.dot_general(a, h_ref[...], (((1,), (0,)), ((), ())),
                            preferred_element_type=jnp.float32))
    pooled = jnp.sum(hn, axis=0) / jnp.float32(n)
    c = jnp.sum(pooled * fcw_ref[_NUM_CLASSES:, 0]) + fcb_ref[0, 0]
    z = jax.lax.dot_general(
        van_ref[...], fcw_ref[: _NUM_CLASSES, :], (((1,), (0,)), ((), ())),
        preferred_element_type=jnp.float32) + c
    beta = jnp.float32(1.1)
    bz = beta * z
    t = (jnp.maximum(bz, 0.0) + jnp.log1p(jnp.exp(-jnp.abs(bz)))) / beta
    out_ref[...] = van_ref[...] / t


_NUM_CLASSES = 10


@jax.jit
def kernel(num_x, cat_x, edge_index, edge_weights, batch, vanilla_out,
           emb_W, conv_W, fc_W, fc_b):
    del batch
    nf = num_x.shape[1]
    tile = 2048
    grid = nf // tile
    h = pl.pallas_call(
        _main_body,
        grid=(grid,),
        in_specs=[
            pl.BlockSpec((_CONT, tile), lambda i: (0, i)),
            pl.BlockSpec((_CATF, tile, _HIDDEN), lambda i: (0, i, 0)),
            pl.BlockSpec((_CATF, _HIDDEN), lambda i: (0, 0)),
            pl.BlockSpec((tile, _HIDDEN), lambda i: (i, 0)),
        ],
        out_specs=pl.BlockSpec((_N_NODES, _HIDDEN), lambda i: (0, 0)),
        out_shape=jax.ShapeDtypeStruct((_N_NODES, _HIDDEN), jnp.float32),
    )(num_x, cat_x, emb_W, conv_W)

    out = pl.pallas_call(
        _epilogue_body,
        out_shape=jax.ShapeDtypeStruct(vanilla_out.shape, jnp.float32),
    )(h, edge_index, edge_weights.reshape(1, -1), vanilla_out, fc_W,
      fc_b.reshape(1, 1))
    return out


# speedup vs baseline: 1.1214x; 1.1056x over previous
import jax
import jax.numpy as jnp
from jax.experimental import pallas as pl
from jax.experimental.pallas import tpu as pltpu

_N_NODES = 39
_HIDDEN = 128
_CONT = 13
_CATF = 26
_NUM_CLASSES = 10
_TILE = 2048


def _build_adjacency(ei_ref, ew_ref, a_ref):
    src = ei_ref[0, :]
    dst = ei_ref[1, :]
    w = ew_ref[0, :]
    e = src.shape[0]
    n = _N_NODES
    node_ids = jax.lax.broadcasted_iota(jnp.int32, (e, n), 1)
    osrc = (src[:, None] == node_ids).astype(jnp.float32)
    odst = (dst[:, None] == node_ids).astype(jnp.float32)
    deg = 1.0 + jnp.sum(odst * w[:, None], axis=0)
    dinv = jnp.where(deg > 0, jax.lax.rsqrt(deg), 0.0)
    dinv_src = jnp.sum(osrc * dinv[None, :], axis=1)
    dinv_dst = jnp.sum(odst * dinv[None, :], axis=1)
    norm = dinv_src * w * dinv_dst
    a = jax.lax.dot_general(
        odst * norm[:, None], osrc, (((0,), (0,)), ((), ())),
        preferred_element_type=jnp.float32)
    rows = jax.lax.broadcasted_iota(jnp.int32, (n, n), 0)
    cols = jax.lax.broadcasted_iota(jnp.int32, (n, n), 1)
    a_ref[...] = a + jnp.where(rows == cols, dinv[:, None] * dinv[None, :], 0.0)


def _body(ei_ref, ew_ref, vanT_ref, fcw_ref, fcb_ref,
          num_ref, cat_ref, embw_ref, convw_ref,
          outT_ref, h_ref, a_ref):
    i = pl.program_id(0)
    ni = pl.num_programs(0)

    emb = jnp.sum(cat_ref[...] * embw_ref[...][:, None, :], axis=2)
    x = jnp.concatenate([num_ref[...], emb], axis=0)
    acc = jax.lax.dot_general(
        x, convw_ref[...], (((1,), (0,)), ((), ())),
        preferred_element_type=jnp.float32)

    @pl.when(i == 0)
    def _():
        h_ref[...] = acc
        _build_adjacency(ei_ref, ew_ref, a_ref)

    @pl.when(i > 0)
    def _():
        h_ref[...] += acc

    @pl.when(i == ni - 1)
    def _():
        hn = jax.nn.relu(
            jax.lax.dot_general(a_ref[...], h_ref[...], (((1,), (0,)), ((), ())),
                                preferred_element_type=jnp.float32))
        pooled = jnp.sum(hn, axis=0, keepdims=True) / jnp.float32(_N_NODES)
        c = jax.lax.dot_general(
            pooled, fcw_ref[_NUM_CLASSES:, :], (((1,), (0,)), ((), ())),
            preferred_element_type=jnp.float32)[0, 0] + fcb_ref[0, 0]
        z = jax.lax.dot_general(
            fcw_ref[:_NUM_CLASSES, :], vanT_ref[...], (((0,), (0,)), ((), ())),
            preferred_element_type=jnp.float32) + c
        beta = jnp.float32(1.1)
        bz = beta * z
        t = (jnp.maximum(bz, 0.0) + jnp.log1p(jnp.exp(-jnp.abs(bz)))) / beta
        outT_ref[...] = vanT_ref[...] / t


@jax.jit
def kernel(num_x, cat_x, edge_index, edge_weights, batch, vanilla_out,
           emb_W, conv_W, fc_W, fc_b):
    del batch
    nf = num_x.shape[1]
    b = vanilla_out.shape[0]
    grid = nf // _TILE
    outT = pl.pallas_call(
        _body,
        grid=(grid,),
        in_specs=[
            pl.BlockSpec((2, edge_index.shape[1]), lambda i: (0, 0)),
            pl.BlockSpec((1, edge_weights.shape[0]), lambda i: (0, 0)),
            pl.BlockSpec((_NUM_CLASSES, b), lambda i: (0, 0)),
            pl.BlockSpec(fc_W.shape, lambda i: (0, 0)),
            pl.BlockSpec((1, 1), lambda i: (0, 0)),
            pl.BlockSpec((_CONT, _TILE), lambda i: (0, i)),
            pl.BlockSpec((_CATF, _TILE, _HIDDEN), lambda i: (0, i, 0)),
            pl.BlockSpec((_CATF, _HIDDEN), lambda i: (0, 0)),
            pl.BlockSpec((_TILE, _HIDDEN), lambda i: (i, 0)),
        ],
        out_specs=pl.BlockSpec((_NUM_CLASSES, b), lambda i: (0, 0)),
        out_shape=jax.ShapeDtypeStruct((_NUM_CLASSES, b), jnp.float32),
        scratch_shapes=[
            pltpu.VMEM((_N_NODES, _HIDDEN), jnp.float32),
            pltpu.VMEM((_N_NODES, _N_NODES), jnp.float32),
        ],
    )(edge_index, edge_weights.reshape(1, -1), vanilla_out.T, fc_W,
      fc_b.reshape(1, 1), num_x, cat_x, emb_W, conv_W)
    return outT.T
